# baseline (speedup 1.0000x reference)
import functools

import jax
import jax.numpy as jnp
from jax import lax
from jax.experimental import pallas as pl
from jax.experimental.pallas import tpu as pltpu
from jax.experimental.pallas import tpu_sc as plsc

_NC = 2
_NS = 16
_LANES = 16


def _sc_dots(inp, tgt2d, emb_table, lin_weight, B, K, D):
    NW = _NC * _NS
    BPW = B // NW
    CB = 64
    NSUB = BPW // CB
    PR = CB * K
    RW = PR // 128

    mesh = plsc.VectorSubcoreMesh(
        core_axis_name="c", subcore_axis_name="s",
        num_cores=_NC, num_subcores=_NS)

    @functools.partial(
        pl.kernel,
        mesh=mesh,
        compiler_params=pltpu.CompilerParams(
            use_tc_tiling_on_sc=False, needs_layout_passes=False),
        out_type=jax.ShapeDtypeStruct((B * K,), jnp.float32),
        scratch_types=[
            pltpu.VMEM((CB,), jnp.int32),
            pltpu.VMEM((PR,), jnp.int32),
            pltpu.VMEM((CB, D), jnp.float32),
            pltpu.VMEM((PR, D), jnp.float32),
            pltpu.VMEM((PR,), jnp.float32),
            pltpu.SemaphoreType.DMA,
        ],
    )
    def sc_kernel(inp_hbm, tgt_hbm, emb_hbm, lin_hbm, out_hbm,
                  idx_v, tgt_v, e_v, t_v, dots_v, sem):
        wid = lax.axis_index("s") * _NC + lax.axis_index("c")
        lanes = lax.iota(jnp.int32, _LANES)

        def sub(s, carry):
            base_b = wid * BPW + s * CB
            pltpu.sync_copy(inp_hbm.at[pl.ds(base_b, CB)], idx_v)
            pltpu.async_copy(emb_hbm.at[idx_v], e_v, sem).wait()
            pltpu.sync_copy(tgt_hbm.at[pl.ds(base_b * K, PR)], tgt_v)
            cps = [
                pltpu.async_copy(lin_hbm.at[tgt_v.at[pl.ds(j * 128, 128)]],
                                 t_v.at[pl.ds(j * 128, 128)], sem)
                for j in range(RW)
            ]
            for c in cps:
                c.wait()

            def grp(g, carry2):
                p = g * _LANES + lanes
                bl = p // K
                acc = jnp.zeros((_LANES,), jnp.float32)
                for d in range(D):
                    dsplat = jnp.full((_LANES,), d, jnp.int32)
                    tv = plsc.load_gather(t_v, [p, dsplat])
                    ev = plsc.load_gather(e_v, [bl, dsplat])
                    acc = acc + tv * ev
                dots_v[pl.ds(g * _LANES, _LANES)] = acc
                return carry2

            lax.fori_loop(0, PR // _LANES, grp, 0)
            pltpu.sync_copy(dots_v, out_hbm.at[pl.ds(base_b * K, PR)])
            return carry

        lax.fori_loop(0, NSUB, sub, 0)

    return sc_kernel(inp, tgt2d, emb_table, lin_weight)


def _tc_loss_sum(dots2d):
    def body(x_ref, o_ref):
        x = x_ref[...]
        o_ref[0, 0] = jnp.sum(jnp.log1p(jnp.exp(-x)))

    return pl.pallas_call(
        body,
        out_shape=jax.ShapeDtypeStruct((1, 1), jnp.float32),
        out_specs=pl.BlockSpec(memory_space=pltpu.SMEM),
    )(dots2d)


def kernel(input, targets, emb_table, lin_weight):
    B, = input.shape
    _, K = targets.shape
    _, D = emb_table.shape
    inp = input.astype(jnp.int32)
    tgt_flat = targets.astype(jnp.int32).reshape(B * K)
    dots = _sc_dots(inp, tgt_flat, emb_table, lin_weight, B, K, D)
    s = _tc_loss_sum(dots.reshape(B * K // 128, 128))
    return s[0, 0] / (B * K)


# speedup vs baseline: 1.3024x; 1.3024x over previous
import functools

import jax
import jax.numpy as jnp
from jax import lax
from jax.experimental import pallas as pl
from jax.experimental.pallas import tpu as pltpu
from jax.experimental.pallas import tpu_sc as plsc

_NC = 2
_NS = 16
_LANES = 16


def _sc_dots(inp, tgt2d, emb_table, lin_weight, B, K, D):
    NW = _NC * _NS
    BPW = B // NW
    CB = 64
    NSUB = BPW // CB
    PR = CB * K
    RW = PR // 128

    mesh = plsc.VectorSubcoreMesh(
        core_axis_name="c", subcore_axis_name="s",
        num_cores=_NC, num_subcores=_NS)

    @functools.partial(
        pl.kernel,
        mesh=mesh,
        compiler_params=pltpu.CompilerParams(
            use_tc_tiling_on_sc=False, needs_layout_passes=False),
        out_type=jax.ShapeDtypeStruct((B * K,), jnp.float32),
        scratch_types=[
            pltpu.VMEM((CB,), jnp.int32),
            pltpu.VMEM((PR,), jnp.int32),
            pltpu.VMEM((CB, D), jnp.float32),
            pltpu.VMEM((PR, D), jnp.float32),
            pltpu.VMEM((PR,), jnp.float32),
            pltpu.VMEM((4 * K, _LANES), jnp.float32),
            pltpu.SemaphoreType.DMA,
        ],
    )
    def sc_kernel(inp_hbm, tgt_hbm, emb_hbm, lin_hbm, out_hbm,
                  idx_v, tgt_v, e_v, t_v, dots_v, macc_v, sem):
        wid = lax.axis_index("s") * _NC + lax.axis_index("c")
        NCH = D // _LANES

        def sub(s, carry):
            base_b = wid * BPW + s * CB
            pltpu.sync_copy(inp_hbm.at[pl.ds(base_b, CB)], idx_v)
            pltpu.async_copy(emb_hbm.at[idx_v], e_v, sem).wait()
            pltpu.sync_copy(tgt_hbm.at[pl.ds(base_b * K, PR)], tgt_v)
            pltpu.async_copy(lin_hbm.at[tgt_v], t_v, sem).wait()

            def bblock(q, carry2):
                b0 = q * 4
                es = [[e_v[b0 + bb, pl.ds(c * _LANES, _LANES)]
                       for c in range(NCH)] for bb in range(4)]
                for bb in range(4):
                    for k in range(K):
                        row = (b0 + bb) * K + k
                        acc = t_v[row, pl.ds(0, _LANES)] * es[bb][0]
                        for c in range(1, NCH):
                            acc = acc + t_v[row, pl.ds(c * _LANES, _LANES)] * es[bb][c]
                        macc_v[bb * K + k] = acc
                for j in range(4 * K // _LANES):
                    rowsel = jnp.arange(_LANES, dtype=jnp.int32) + j * _LANES
                    dot = plsc.load_gather(
                        macc_v, [rowsel, jnp.full((_LANES,), 0, jnp.int32)])
                    for c in range(1, _LANES):
                        dot = dot + plsc.load_gather(
                            macc_v, [rowsel, jnp.full((_LANES,), c, jnp.int32)])
                    dots_v[pl.ds(q * 4 * K + j * _LANES, _LANES)] = dot
                return carry2

            lax.fori_loop(0, CB // 4, bblock, 0)
            pltpu.sync_copy(dots_v, out_hbm.at[pl.ds(base_b * K, PR)])
            return carry

        lax.fori_loop(0, NSUB, sub, 0)

    return sc_kernel(inp, tgt2d, emb_table, lin_weight)


def _tc_loss_sum(dots2d):
    def body(x_ref, o_ref):
        x = x_ref[...]
        o_ref[0, 0] = jnp.sum(jnp.log1p(jnp.exp(-x)))

    return pl.pallas_call(
        body,
        out_shape=jax.ShapeDtypeStruct((1, 1), jnp.float32),
        out_specs=pl.BlockSpec(memory_space=pltpu.SMEM),
    )(dots2d)


def kernel(input, targets, emb_table, lin_weight):
    B, = input.shape
    _, K = targets.shape
    _, D = emb_table.shape
    inp = input.astype(jnp.int32)
    tgt_flat = targets.astype(jnp.int32).reshape(B * K)
    dots = _sc_dots(inp, tgt_flat, emb_table, lin_weight, B, K, D)
    s = _tc_loss_sum(dots.reshape(B * K // 128, 128))
    return s[0, 0] / (B * K)
